# baseline (speedup 1.0000x reference)
import functools

import jax
import jax.numpy as jnp
from jax import lax
from jax.experimental import pallas as pl
from jax.experimental.pallas import tpu as pltpu
from jax.experimental.pallas import tpu_sc as plsc

K = 5

_NC, _NS, _L = 2, 16, 16
_NW = _NC * _NS


_WIN = 128


def _make_tc_windows(B, N, RB):
    assert B % RB == 0

    def body(lab_ref, pred_ref, out_ref, sem):
        r = pl.program_id(0)

        def dma(i):
            labi = lab_ref[i, 0]
            c0 = pl.multiple_of(jnp.minimum((labi >> 7) << 7, N - _WIN),
                                _WIN)
            r8 = pl.multiple_of(r * RB + ((i >> 3) << 3), 8)
            return pltpu.make_async_copy(
                pred_ref.at[pl.ds(r8, 8), pl.ds(c0, _WIN)],
                out_ref.at[i], sem)

        def issue(i, carry):
            dma(i).start()
            return carry

        lax.fori_loop(0, RB, issue, 0)

        def drain(i, carry):
            dma(i).wait()
            return carry

        lax.fori_loop(0, RB, drain, 0)

    return pl.pallas_call(
        body,
        grid=(B // RB,),
        in_specs=[
            pl.BlockSpec((RB, 1), lambda r: (r, 0),
                         memory_space=pltpu.MemorySpace.SMEM),
            pl.BlockSpec(memory_space=pltpu.MemorySpace.HBM),
        ],
        out_specs=pl.BlockSpec((RB, 8, _WIN), lambda r: (r, 0, 0)),
        out_shape=jax.ShapeDtypeStruct((B, 8, _WIN), jnp.float32),
        scratch_shapes=[pltpu.SemaphoreType.DMA],
    )


def _make_sc_extract(B, N):
    assert B % (_NW * 128) == 0
    bpw = B // _NW
    nseg = bpw // 128
    mesh = plsc.VectorSubcoreMesh(core_axis_name="c", subcore_axis_name="s")

    @functools.partial(
        pl.kernel,
        mesh=mesh,
        out_type=jax.ShapeDtypeStruct((B,), jnp.float32),
        scratch_types=[
            pltpu.VMEM((bpw,), jnp.int32),
            pltpu.VMEM((nseg, 128), jnp.int32),
            pltpu.VMEM((bpw,), jnp.float32),
            pltpu.SemaphoreType.DMA,
        ],
    )
    def sc_extract(win_hbm, lab_hbm, out_hbm, lab_v, idx_v, val_v, sem):
        wid = lax.axis_index("s") * _NC + lax.axis_index("c")
        base = wid * bpw
        pltpu.sync_copy(lab_hbm.at[pl.ds(base, bpw)], lab_v)
        for t in range(bpw // _L):
            s = t * _L
            l = lab_v[pl.ds(s, _L)]
            c0 = jnp.minimum((l >> 7) << 7, N - _WIN)
            row = base + s + lax.iota(jnp.int32, _L)
            idx_v[s // 128, pl.ds(s % 128, _L)] = \
                row * (8 * _WIN) + (row & 7) * _WIN + (l - c0)
        for j in range(nseg):
            pltpu.async_copy(
                win_hbm.at[idx_v.at[j]],
                val_v.at[pl.ds(j * 128, 128)],
                sem,
            ).wait()
        pltpu.sync_copy(val_v, out_hbm.at[pl.ds(base, bpw)])

    return sc_extract


def _count_block(x, v, lab, RB, N):
    bv = lax.bitcast_convert_type(v, jnp.int32)
    bvd = jnp.where(bv > 0, bv - 1,
                    jnp.where(bv == 0, jnp.int32(-2147483647), bv + 1))
    vd = lax.bitcast_convert_type(bvd, jnp.float32)
    colv = lax.broadcasted_iota(jnp.int32, (RB, N), 1)
    t = jnp.where(colv < lab, vd, v)
    cnt = jnp.sum(x > t, axis=1, keepdims=True).astype(jnp.int32)
    return jnp.sum((cnt < K).astype(jnp.float32))


def _make_tc_count(B, N, RB):
    assert B % (2 * RB) == 0
    H = B // 2
    rgrid = H // RB

    def body(v1_ref, lab1_ref, v2_ref, lab2_ref, p1_ref, p2_ref,
             out_ref, tot_ref):
        r = pl.program_id(0)
        good = (_count_block(p1_ref[...], v1_ref[...], lab1_ref[...], RB, N)
                + _count_block(p2_ref[...], v2_ref[...], lab2_ref[...],
                               RB, N))
        prev = jnp.where(r == 0, jnp.float32(0.0), tot_ref[0])
        tot = prev + good
        tot_ref[0] = tot

        @pl.when(r == rgrid - 1)
        def _():
            out_ref[0, 0] = tot * jnp.float32(100.0 / B)

    return pl.pallas_call(
        body,
        grid=(rgrid,),
        in_specs=[
            pl.BlockSpec((RB, 1), lambda r: (r, 0)),
            pl.BlockSpec((RB, 1), lambda r: (r, 0)),
            pl.BlockSpec((RB, 1), lambda r: (r, 0)),
            pl.BlockSpec((RB, 1), lambda r: (r, 0)),
            pl.BlockSpec((RB, N), lambda r: (r, 0)),
            pl.BlockSpec((RB, N), lambda r: (r, 0)),
        ],
        out_specs=pl.BlockSpec((1, 1), lambda r: (0, 0),
                               memory_space=pltpu.SMEM),
        out_shape=jax.ShapeDtypeStruct((1, 1), jnp.float32),
        scratch_shapes=[
            pltpu.SMEM((1,), jnp.float32),
        ],
        compiler_params=pltpu.CompilerParams(
            dimension_semantics=("arbitrary",)),
    )


def kernel(pred, lab):
    B, N = pred.shape
    lab32 = lab.astype(jnp.int32)
    win = _make_tc_windows(B, N, 256)(lab32.reshape(B, 1), pred)
    v = _make_sc_extract(B, N)(win.reshape(B * 8 * _WIN), lab32)
    H = B // 2
    v2d = v.reshape(B, 1)
    lab2d = lab32.reshape(B, 1)
    out = _make_tc_count(B, N, 32)(
        v2d[:H], lab2d[:H], v2d[H:], lab2d[H:], pred[:H], pred[H:])
    return out[0, 0]


# speedup vs baseline: 1.5100x; 1.5100x over previous
import functools

import jax
import jax.numpy as jnp
from jax import lax
from jax.experimental import pallas as pl
from jax.experimental.pallas import tpu as pltpu
from jax.experimental.pallas import tpu_sc as plsc

K = 5

_NC, _NS, _L = 2, 16, 16
_NW = _NC * _NS


_WIN = 128


def _make_tc_windows(B, N, RB):
    assert B % RB == 0

    def body(lab_ref, pred_ref, out_ref, sem):
        r = pl.program_id(0)

        def dma(i):
            labi = lab_ref[i, 0]
            c0 = pl.multiple_of(jnp.minimum((labi >> 7) << 7, N - _WIN),
                                _WIN)
            r8 = pl.multiple_of(r * RB + ((i >> 3) << 3), 8)
            return pltpu.make_async_copy(
                pred_ref.at[pl.ds(r8, 8), pl.ds(c0, _WIN)],
                out_ref.at[i], sem)

        def issue(i, carry):
            dma(i).start()
            return carry

        lax.fori_loop(0, RB, issue, 0)

        def drain(i, carry):
            dma(i).wait()
            return carry

        lax.fori_loop(0, RB, drain, 0)

    return pl.pallas_call(
        body,
        grid=(B // RB,),
        in_specs=[
            pl.BlockSpec((RB, 1), lambda r: (r, 0),
                         memory_space=pltpu.MemorySpace.SMEM),
            pl.BlockSpec(memory_space=pltpu.MemorySpace.HBM),
        ],
        out_specs=pl.BlockSpec((RB, 8, _WIN), lambda r: (r, 0, 0)),
        out_shape=jax.ShapeDtypeStruct((B, 8, _WIN), jnp.float32),
        scratch_shapes=[pltpu.SemaphoreType.DMA],
    )


def _make_sc_extract(B, N):
    assert B % (_NW * 128) == 0
    bpw = B // _NW
    nseg = bpw // 128
    mesh = plsc.VectorSubcoreMesh(core_axis_name="c", subcore_axis_name="s")

    @functools.partial(
        pl.kernel,
        mesh=mesh,
        out_type=jax.ShapeDtypeStruct((B,), jnp.float32),
        scratch_types=[
            pltpu.VMEM((bpw,), jnp.int32),
            pltpu.VMEM((nseg, 128), jnp.int32),
            pltpu.VMEM((bpw,), jnp.float32),
            pltpu.SemaphoreType.DMA,
        ],
    )
    def sc_extract(win_hbm, lab_hbm, out_hbm, lab_v, idx_v, val_v, sem):
        wid = lax.axis_index("s") * _NC + lax.axis_index("c")
        base = wid * bpw
        pltpu.sync_copy(lab_hbm.at[pl.ds(base, bpw)], lab_v)
        for t in range(bpw // _L):
            s = t * _L
            l = lab_v[pl.ds(s, _L)]
            c0 = jnp.minimum((l >> 7) << 7, N - _WIN)
            row = base + s + lax.iota(jnp.int32, _L)
            idx_v[s // 128, pl.ds(s % 128, _L)] = \
                row * (8 * _WIN) + (row & 7) * _WIN + (l - c0)
        for j in range(nseg):
            pltpu.async_copy(
                win_hbm.at[idx_v.at[j]],
                val_v.at[pl.ds(j * 128, 128)],
                sem,
            ).wait()
        pltpu.sync_copy(val_v, out_hbm.at[pl.ds(base, bpw)])

    return sc_extract


def _count_block(x, v, lab, RB, N):
    bv = lax.bitcast_convert_type(v, jnp.int32)
    bvd = jnp.where(bv > 0, bv - 1,
                    jnp.where(bv == 0, jnp.int32(-2147483647), bv + 1))
    vd = lax.bitcast_convert_type(bvd, jnp.float32)
    colv = lax.broadcasted_iota(jnp.int32, (RB, N), 1)
    t = jnp.where(colv < lab, vd, v)
    cnt = jnp.sum(x > t, axis=1, keepdims=True).astype(jnp.int32)
    return jnp.sum((cnt < K).astype(jnp.float32))


def _make_tc_count(B, N, RB, BTC):
    assert BTC % RB == 0
    rgrid = BTC // RB

    def body(v_ref, lab_ref, pred_ref, out_ref, tot_ref):
        r = pl.program_id(0)
        good = _count_block(pred_ref[...], v_ref[...], lab_ref[...], RB, N)
        prev = jnp.where(r == 0, jnp.float32(0.0), tot_ref[0])
        tot = prev + good
        tot_ref[0] = tot

        @pl.when(r == rgrid - 1)
        def _():
            out_ref[0, 0] = tot

    return pl.pallas_call(
        body,
        grid=(rgrid,),
        in_specs=[
            pl.BlockSpec((RB, 1), lambda r: (r, 0)),
            pl.BlockSpec((RB, 1), lambda r: (r, 0)),
            pl.BlockSpec((RB, N), lambda r: (r, 0)),
        ],
        out_specs=pl.BlockSpec((1, 1), lambda r: (0, 0),
                               memory_space=pltpu.SMEM),
        out_shape=jax.ShapeDtypeStruct((1, 1), jnp.float32),
        scratch_shapes=[
            pltpu.SMEM((1,), jnp.float32),
        ],
        compiler_params=pltpu.CompilerParams(
            dimension_semantics=("arbitrary",)),
    )


def _make_tc_count2(B, N, RB):
    assert B % (2 * RB) == 0
    rgrid = B // (2 * RB)

    def body(v1, lab1, v2, lab2, p1, p2, out_ref, tot_ref):
        r = pl.program_id(0)
        good = (_count_block(p1[...], v1[...], lab1[...], RB, N)
                + _count_block(p2[...], v2[...], lab2[...], RB, N))
        prev = jnp.where(r == 0, jnp.float32(0.0), tot_ref[0])
        tot = prev + good
        tot_ref[0] = tot

        @pl.when(r == rgrid - 1)
        def _():
            out_ref[0, 0] = tot

    return pl.pallas_call(
        body,
        grid=(rgrid,),
        in_specs=[
            pl.BlockSpec((RB, 1), lambda r: (2 * r, 0)),
            pl.BlockSpec((RB, 1), lambda r: (2 * r, 0)),
            pl.BlockSpec((RB, 1), lambda r: (2 * r + 1, 0)),
            pl.BlockSpec((RB, 1), lambda r: (2 * r + 1, 0)),
            pl.BlockSpec((RB, N), lambda r: (2 * r, 0)),
            pl.BlockSpec((RB, N), lambda r: (2 * r + 1, 0)),
        ],
        out_specs=pl.BlockSpec((1, 1), lambda r: (0, 0),
                               memory_space=pltpu.SMEM),
        out_shape=jax.ShapeDtypeStruct((1, 1), jnp.float32),
        scratch_shapes=[
            pltpu.SMEM((1,), jnp.float32),
        ],
        compiler_params=pltpu.CompilerParams(
            dimension_semantics=("arbitrary",)),
    )


def kernel(pred, lab):
    B, N = pred.shape
    lab32 = lab.astype(jnp.int32)
    win = _make_tc_windows(B, N, 256)(lab32.reshape(B, 1), pred)
    v = _make_sc_extract(B, N)(win.reshape(B * 8 * _WIN), lab32)
    v2d = v.reshape(B, 1)
    lab2d = lab32.reshape(B, 1)
    out = _make_tc_count2(B, N, 32)(v2d, lab2d, v2d, lab2d, pred, pred)
    return out[0, 0] * jnp.float32(100.0 / B)
